# baseline (speedup 1.0000x reference)
import functools

import jax
import jax.numpy as jnp
from jax.experimental import pallas as pl


def _ceil_div(a, b):
    return -(-a // b)


def _lin_body(x_ref, w_ref, b_ref, o_ref):
    o_ref[...] = x_ref[...] @ w_ref[...] + b_ref[...]


def _lin(x, W, b, bm):
    n, d = x.shape
    dout = W.shape[1]
    g = _ceil_div(n, bm)
    npad = g * bm
    if npad != n:
        x = jnp.pad(x, ((0, npad - n), (0, 0)))
    out = pl.pallas_call(
        _lin_body,
        grid=(g,),
        in_specs=[
            pl.BlockSpec((bm, d), lambda i: (i, 0)),
            pl.BlockSpec((d, dout), lambda i: (0, 0)),
            pl.BlockSpec((1, dout), lambda i: (0, 0)),
        ],
        out_specs=pl.BlockSpec((bm, dout), lambda i: (i, 0)),
        out_shape=jax.ShapeDtypeStruct((npad, dout), x.dtype),
    )(x, W, b.reshape(1, -1))
    return out[:n]


def _edge_mlp_body(xi_ref, xj_ref, wa_ref, wb_ref, b1_ref, w2_ref, b2_ref, o_ref):
    xi = xi_ref[...]
    xj = xj_ref[...]
    h = xi @ wa_ref[...] + (xj - xi) @ wb_ref[...] + b1_ref[...]
    h = jnp.maximum(h, 0.0)
    o_ref[...] = h @ w2_ref[...] + b2_ref[...]


def _edge_mlp(xi, xj, W1, b1, W2, b2, bm):
    e, d = xi.shape
    dmid = W1.shape[1]
    dout = W2.shape[1]
    Wa = W1[:d]
    Wb = W1[d:]
    g = _ceil_div(e, bm)
    epad = g * bm
    if epad != e:
        xi = jnp.pad(xi, ((0, epad - e), (0, 0)))
        xj = jnp.pad(xj, ((0, epad - e), (0, 0)))
    out = pl.pallas_call(
        _edge_mlp_body,
        grid=(g,),
        in_specs=[
            pl.BlockSpec((bm, d), lambda i: (i, 0)),
            pl.BlockSpec((bm, d), lambda i: (i, 0)),
            pl.BlockSpec((d, dmid), lambda i: (0, 0)),
            pl.BlockSpec((d, dmid), lambda i: (0, 0)),
            pl.BlockSpec((1, dmid), lambda i: (0, 0)),
            pl.BlockSpec((dmid, dout), lambda i: (0, 0)),
            pl.BlockSpec((1, dout), lambda i: (0, 0)),
        ],
        out_specs=pl.BlockSpec((bm, dout), lambda i: (i, 0)),
        out_shape=jax.ShapeDtypeStruct((epad, dout), xi.dtype),
    )(xi, xj, Wa, Wb, b1.reshape(1, -1), W2, b2.reshape(1, -1))
    return out[:e]


def _sage_comb_body(a_ref, wl_ref, bl_ref, x_ref, wr_ref, o_ref):
    o_ref[...] = a_ref[...] @ wl_ref[...] + bl_ref[...] + x_ref[...] @ wr_ref[...]


def _sage_comb(avg, xd, Wl, bl, Wr, bm):
    n, dl = avg.shape
    dr = xd.shape[1]
    dout = Wl.shape[1]
    g = _ceil_div(n, bm)
    npad = g * bm
    if npad != n:
        avg = jnp.pad(avg, ((0, npad - n), (0, 0)))
        xd = jnp.pad(xd, ((0, npad - n), (0, 0)))
    out = pl.pallas_call(
        _sage_comb_body,
        grid=(g,),
        in_specs=[
            pl.BlockSpec((bm, dl), lambda i: (i, 0)),
            pl.BlockSpec((dl, dout), lambda i: (0, 0)),
            pl.BlockSpec((1, dout), lambda i: (0, 0)),
            pl.BlockSpec((bm, dr), lambda i: (i, 0)),
            pl.BlockSpec((dr, dout), lambda i: (0, 0)),
        ],
        out_specs=pl.BlockSpec((bm, dout), lambda i: (i, 0)),
        out_shape=jax.ShapeDtypeStruct((npad, dout), avg.dtype),
    )(avg, Wl, bl.reshape(1, -1), xd, Wr)
    return out[:n]


def _bn(x, g, b):
    m = x.mean(0)
    v = x.var(0)
    return (x - m) / jnp.sqrt(v + 1e-5) * g + b


def _edge_conv(x, ei, W1, b1, W2, b2, bm):
    xi = x[ei[1]]
    xj = x[ei[0]]
    m = _edge_mlp(xi, xj, W1, b1, W2, b2, bm)
    out = jax.ops.segment_max(m, ei[1], num_segments=x.shape[0])
    return jnp.where(jnp.isfinite(out), out, 0.0)


def _sage(xs, xd, ei, Wl, bl, Wr, bm):
    s = jax.ops.segment_sum(xs[ei[0]], ei[1], num_segments=xd.shape[0])
    c = jax.ops.segment_sum(jnp.ones((ei.shape[1], 1), xs.dtype), ei[1],
                            num_segments=xd.shape[0])
    avg = s / jnp.maximum(c, 1.0)
    return _sage_comb(avg, xd, Wl, bl, Wr, bm)


def _forward(x_o, x_t, p, ei_oo, ei_to, ei_tt):
    BM_O = 2000
    BM_T = 1000
    BM_EOO = 4000
    BM_ETT = 1000

    h_o = jax.nn.relu(_bn(_lin(x_o, p['W011'], p['b011'], BM_O), p['g01'], p['be01']))
    h_t = jax.nn.relu(_bn(_lin(x_t, p['W01t'], p['b01t'], BM_T), p['g01t'], p['be01t']))
    for i in (1, 2, 3):
        o = (_edge_conv(h_o, ei_oo, p['ec_oo%d_W1' % i], p['ec_oo%d_b1' % i],
                        p['ec_oo%d_W2' % i], p['ec_oo%d_b2' % i], BM_EOO)
             + _sage(h_t, h_o, ei_to, p['sg_to%d_Wl' % i], p['sg_to%d_bl' % i],
                     p['sg_to%d_Wr' % i], BM_O))
        t = _edge_conv(h_t, ei_tt, p['ec_tt%d_W1' % i], p['ec_tt%d_b1' % i],
                       p['ec_tt%d_W2' % i], p['ec_tt%d_b2' % i], BM_ETT)
        h_o = jax.nn.relu(_bn(o, p['g1%d' % i], p['be1%d' % i]))
        h_t = jax.nn.relu(_bn(t, p['g1%dt' % i], p['be1%dt' % i]))
    o = (_sage(h_o, h_o, ei_oo, p['sg21_oo_Wl'], p['sg21_oo_bl'], p['sg21_oo_Wr'], BM_O)
         + _sage(h_t, h_o, ei_to, p['sg21_to_Wl'], p['sg21_to_bl'], p['sg21_to_Wr'], BM_O))
    t = _sage(h_t, h_t, ei_tt, p['sg21_tt_Wl'], p['sg21_tt_bl'], p['sg21_tt_Wr'], BM_T)
    h_o = jax.nn.relu(_bn(o, p['g21'], p['be21']))
    h_t = jax.nn.relu(_bn(t, p['g21t'], p['be21t']))
    outs = []
    for k in (1, 2, 3):
        ok = (_sage(h_o, h_o, ei_oo, p['sg3%d_oo_Wl' % k], p['sg3%d_oo_bl' % k],
                    p['sg3%d_oo_Wr' % k], BM_O)
              + _sage(h_t, h_o, ei_to, p['sg3%d_to_Wl' % k], p['sg3%d_to_bl' % k],
                      p['sg3%d_to_Wr' % k], BM_O))
        outs.append(ok)
    return jnp.stack(outs)


@jax.jit
def _kernel_impl(x_o, x_t, edge_index_oo, edge_index_to, edge_index_tt, params):
    return _forward(x_o, x_t, params, edge_index_oo, edge_index_to, edge_index_tt)


def kernel(x_o, x_t, edge_index_oo, edge_index_to, edge_index_tt, edge_attr_oo, params):
    del edge_attr_oo
    return _kernel_impl(x_o, x_t, edge_index_oo, edge_index_to, edge_index_tt, params)


# speedup vs baseline: 1.7120x; 1.7120x over previous
import functools

import jax
import jax.numpy as jnp
from jax.experimental import pallas as pl


def _ceil_div(a, b):
    return -(-a // b)


def _lin_body(x_ref, w_ref, b_ref, o_ref):
    o_ref[...] = x_ref[...] @ w_ref[...] + b_ref[...]


def _lin(x, W, b, bm):
    n, d = x.shape
    dout = W.shape[1]
    g = _ceil_div(n, bm)
    npad = g * bm
    if npad != n:
        x = jnp.pad(x, ((0, npad - n), (0, 0)))
    out = pl.pallas_call(
        _lin_body,
        grid=(g,),
        in_specs=[
            pl.BlockSpec((bm, d), lambda i: (i, 0)),
            pl.BlockSpec((d, dout), lambda i: (0, 0)),
            pl.BlockSpec((1, dout), lambda i: (0, 0)),
        ],
        out_specs=pl.BlockSpec((bm, dout), lambda i: (i, 0)),
        out_shape=jax.ShapeDtypeStruct((npad, dout), x.dtype),
    )(x, W, b.reshape(1, -1))
    return out[:n]


def _edge_mlp_body(xi_ref, xj_ref, wa_ref, wb_ref, b1_ref, w2_ref, b2_ref, o_ref):
    xi = xi_ref[...]
    xj = xj_ref[...]
    h = xi @ wa_ref[...] + (xj - xi) @ wb_ref[...] + b1_ref[...]
    h = jnp.maximum(h, 0.0)
    o_ref[...] = h @ w2_ref[...] + b2_ref[...]


def _edge_mlp(xi, xj, W1, b1, W2, b2, bm):
    e, d = xi.shape
    dmid = W1.shape[1]
    dout = W2.shape[1]
    Wa = W1[:d]
    Wb = W1[d:]
    g = _ceil_div(e, bm)
    epad = g * bm
    if epad != e:
        xi = jnp.pad(xi, ((0, epad - e), (0, 0)))
        xj = jnp.pad(xj, ((0, epad - e), (0, 0)))
    out = pl.pallas_call(
        _edge_mlp_body,
        grid=(g,),
        in_specs=[
            pl.BlockSpec((bm, d), lambda i: (i, 0)),
            pl.BlockSpec((bm, d), lambda i: (i, 0)),
            pl.BlockSpec((d, dmid), lambda i: (0, 0)),
            pl.BlockSpec((d, dmid), lambda i: (0, 0)),
            pl.BlockSpec((1, dmid), lambda i: (0, 0)),
            pl.BlockSpec((dmid, dout), lambda i: (0, 0)),
            pl.BlockSpec((1, dout), lambda i: (0, 0)),
        ],
        out_specs=pl.BlockSpec((bm, dout), lambda i: (i, 0)),
        out_shape=jax.ShapeDtypeStruct((epad, dout), xi.dtype),
    )(xi, xj, Wa, Wb, b1.reshape(1, -1), W2, b2.reshape(1, -1))
    return out[:e]


def _sage_comb_body(a_ref, wl_ref, bl_ref, x_ref, wr_ref, o_ref):
    o_ref[...] = a_ref[...] @ wl_ref[...] + bl_ref[...] + x_ref[...] @ wr_ref[...]


def _sage_comb(avg, xd, Wl, bl, Wr, bm):
    n, dl = avg.shape
    dr = xd.shape[1]
    dout = Wl.shape[1]
    g = _ceil_div(n, bm)
    npad = g * bm
    if npad != n:
        avg = jnp.pad(avg, ((0, npad - n), (0, 0)))
        xd = jnp.pad(xd, ((0, npad - n), (0, 0)))
    out = pl.pallas_call(
        _sage_comb_body,
        grid=(g,),
        in_specs=[
            pl.BlockSpec((bm, dl), lambda i: (i, 0)),
            pl.BlockSpec((dl, dout), lambda i: (0, 0)),
            pl.BlockSpec((1, dout), lambda i: (0, 0)),
            pl.BlockSpec((bm, dr), lambda i: (i, 0)),
            pl.BlockSpec((dr, dout), lambda i: (0, 0)),
        ],
        out_specs=pl.BlockSpec((bm, dout), lambda i: (i, 0)),
        out_shape=jax.ShapeDtypeStruct((npad, dout), avg.dtype),
    )(avg, Wl, bl.reshape(1, -1), xd, Wr)
    return out[:n]


def _addmm_body(a_ref, bl_ref, x_ref, wr_ref, o_ref):
    o_ref[...] = a_ref[...] + bl_ref[...] + x_ref[...] @ wr_ref[...]


def _addmm(a, bl, xd, Wr, bm):
    n, dout = a.shape
    dr = xd.shape[1]
    g = _ceil_div(n, bm)
    npad = g * bm
    if npad != n:
        a = jnp.pad(a, ((0, npad - n), (0, 0)))
        xd = jnp.pad(xd, ((0, npad - n), (0, 0)))
    out = pl.pallas_call(
        _addmm_body,
        grid=(g,),
        in_specs=[
            pl.BlockSpec((bm, dout), lambda i: (i, 0)),
            pl.BlockSpec((1, dout), lambda i: (0, 0)),
            pl.BlockSpec((bm, dr), lambda i: (i, 0)),
            pl.BlockSpec((dr, dout), lambda i: (0, 0)),
        ],
        out_specs=pl.BlockSpec((bm, dout), lambda i: (i, 0)),
        out_shape=jax.ShapeDtypeStruct((npad, dout), a.dtype),
    )(a, bl.reshape(1, -1), xd, Wr)
    return out[:n]


def _bn(x, g, b):
    m = x.mean(0)
    v = x.var(0)
    return (x - m) / jnp.sqrt(v + 1e-5) * g + b


def _edge_conv(x, ei, W1, b1, W2, b2, bm):
    xi = x[ei[1]]
    xj = x[ei[0]]
    m = _edge_mlp(xi, xj, W1, b1, W2, b2, bm)
    out = jax.ops.segment_max(m, ei[1], num_segments=x.shape[0])
    return jnp.where(jnp.isfinite(out), out, 0.0)


def _sage(xs, xd, ei, Wl, bl, Wr, bm, bms=None):
    c = jax.ops.segment_sum(jnp.ones((ei.shape[1], 1), xs.dtype), ei[1],
                            num_segments=xd.shape[0])
    if xs.shape[1] > Wl.shape[1]:
        y = _lin(xs, Wl, jnp.zeros((Wl.shape[1],), xs.dtype), bms)
        s = jax.ops.segment_sum(y[ei[0]], ei[1], num_segments=xd.shape[0])
        avg = s / jnp.maximum(c, 1.0)
        return _addmm(avg, bl, xd, Wr, bm)
    s = jax.ops.segment_sum(xs[ei[0]], ei[1], num_segments=xd.shape[0])
    avg = s / jnp.maximum(c, 1.0)
    return _sage_comb(avg, xd, Wl, bl, Wr, bm)


def _forward(x_o, x_t, p, ei_oo, ei_to, ei_tt):
    BM_O = 2000
    BM_T = 1000
    BM_EOO = 4000
    BM_ETT = 1000

    h_o = jax.nn.relu(_bn(_lin(x_o, p['W011'], p['b011'], BM_O), p['g01'], p['be01']))
    h_t = jax.nn.relu(_bn(_lin(x_t, p['W01t'], p['b01t'], BM_T), p['g01t'], p['be01t']))
    for i in (1, 2, 3):
        o = (_edge_conv(h_o, ei_oo, p['ec_oo%d_W1' % i], p['ec_oo%d_b1' % i],
                        p['ec_oo%d_W2' % i], p['ec_oo%d_b2' % i], BM_EOO)
             + _sage(h_t, h_o, ei_to, p['sg_to%d_Wl' % i], p['sg_to%d_bl' % i],
                     p['sg_to%d_Wr' % i], BM_O, BM_T))
        t = _edge_conv(h_t, ei_tt, p['ec_tt%d_W1' % i], p['ec_tt%d_b1' % i],
                       p['ec_tt%d_W2' % i], p['ec_tt%d_b2' % i], BM_ETT)
        h_o = jax.nn.relu(_bn(o, p['g1%d' % i], p['be1%d' % i]))
        h_t = jax.nn.relu(_bn(t, p['g1%dt' % i], p['be1%dt' % i]))
    o = (_sage(h_o, h_o, ei_oo, p['sg21_oo_Wl'], p['sg21_oo_bl'], p['sg21_oo_Wr'], BM_O)
         + _sage(h_t, h_o, ei_to, p['sg21_to_Wl'], p['sg21_to_bl'], p['sg21_to_Wr'], BM_O, BM_T))
    t = _sage(h_t, h_t, ei_tt, p['sg21_tt_Wl'], p['sg21_tt_bl'], p['sg21_tt_Wr'], BM_T, BM_T)
    h_o = jax.nn.relu(_bn(o, p['g21'], p['be21']))
    h_t = jax.nn.relu(_bn(t, p['g21t'], p['be21t']))
    outs = []
    for k in (1, 2, 3):
        ok = (_sage(h_o, h_o, ei_oo, p['sg3%d_oo_Wl' % k], p['sg3%d_oo_bl' % k],
                    p['sg3%d_oo_Wr' % k], BM_O)
              + _sage(h_t, h_o, ei_to, p['sg3%d_to_Wl' % k], p['sg3%d_to_bl' % k],
                      p['sg3%d_to_Wr' % k], BM_O))
        outs.append(ok)
    return jnp.stack(outs)


@jax.jit
def _kernel_impl(x_o, x_t, edge_index_oo, edge_index_to, edge_index_tt, params):
    return _forward(x_o, x_t, params, edge_index_oo, edge_index_to, edge_index_tt)


def kernel(x_o, x_t, edge_index_oo, edge_index_to, edge_index_tt, edge_attr_oo, params):
    del edge_attr_oo
    return _kernel_impl(x_o, x_t, edge_index_oo, edge_index_to, edge_index_tt, params)
